# baseline (speedup 1.0000x reference)
import functools

import jax
import jax.numpy as jnp
from jax import lax
from jax.experimental import pallas as pl
from jax.experimental.pallas import tpu as pltpu

N_USERS = 4096
N_ITEMS = 65536
HIDDEN = 128
HOP = 3
N_NODES = N_USERS + N_ITEMS

_ROW_TILE = 512


def _matmul_body(m_ref, c_ref, o_ref):
    o_ref[...] = jnp.dot(m_ref[...], c_ref[...],
                         preferred_element_type=jnp.float32)


def _dense_hop(mat, cur):
    n = mat.shape[0]
    grid = (n // _ROW_TILE,)
    return pl.pallas_call(
        _matmul_body,
        grid=grid,
        in_specs=[
            pl.BlockSpec((_ROW_TILE, n), lambda i: (i, 0)),
            pl.BlockSpec((n, HIDDEN), lambda i: (0, 0)),
        ],
        out_specs=pl.BlockSpec((_ROW_TILE, HIDDEN), lambda i: (i, 0)),
        out_shape=jax.ShapeDtypeStruct((n, HIDDEN), jnp.float32),
    )(mat, cur)


def _gcn_dense(adj, ue):
    acc = ue
    c = ue
    for _ in range(HOP):
        c = _dense_hop(adj, c)
        acc = acc + c
    return acc * (1.0 / (HOP + 1))


def kernel(users, pos, neg, user_embs, item_embs, social_mat, sharing_mat,
           A_rows, A_cols, A_vals):
    all_emb = jnp.concatenate([user_embs, item_embs], axis=0)
    acc = all_emb
    cur = all_emb
    for _ in range(HOP):
        msgs = A_vals[:, None] * cur[A_cols]
        cur = jax.ops.segment_sum(msgs, A_rows, num_segments=N_NODES)
        acc = acc + cur
    light_out = acc * (1.0 / (HOP + 1))
    rec_user_embs = light_out[:N_USERS]
    rec_item_embs = light_out[N_USERS:]

    sharing_view_embs = _gcn_dense(sharing_mat, user_embs)
    friend_view_embs = _gcn_dense(social_mat, user_embs)

    users_emb = rec_user_embs[users]
    pos_emb = rec_item_embs[pos]
    neg_emb = rec_item_embs[neg]
    users_emb_ego = user_embs[users]
    pos_emb_ego = item_embs[pos]
    neg_emb_ego = item_embs[neg]
    return (users_emb, pos_emb, neg_emb, users_emb_ego, pos_emb_ego,
            neg_emb_ego, sharing_view_embs, friend_view_embs)
